# Initial kernel scaffold; baseline (speedup 1.0000x reference)
import jax
import jax.numpy as jnp
from jax.experimental import pallas as pl


def kernel(x, edge_index, edge_pairs, W1l, b1l, W1r, W2l, b2l, W2r, Wp1, bp1, Wp2, bp2):
    raise NotImplementedError("write your pallas kernel here")


# speedup vs baseline: 4.2648x; 4.2648x over previous
import functools

import jax
import jax.numpy as jnp
from jax import lax
from jax.experimental import pallas as pl
from jax.experimental.pallas import tpu as pltpu
from jax.experimental.pallas import tpu_sc as plsc

N = 10000
N_PAD = 10240
D = 128
H = 128
CHUNK = 80


def _wid(nc):
    return lax.axis_index("s") * nc + lax.axis_index("c")


def _make_seg_kernel(num_edges, with_cnt, nc, ns):
    nw = nc * ns
    rows_per_tile = N_PAD // ns
    n_chunks = num_edges // (nw * CHUNK)
    edges_per_tile = num_edges // nw

    out_type = [jax.ShapeDtypeStruct((nc, N_PAD, D), jnp.float32)]
    if with_cnt:
        out_type.append(jax.ShapeDtypeStruct((nc, N_PAD, D), jnp.float32))
    scratch = [
        pltpu.VMEM((CHUNK,), jnp.int32),
        pltpu.VMEM((CHUNK,), jnp.int32),
        pltpu.VMEM((CHUNK, D), jnp.float32),
        pltpu.VMEM_SHARED((N_PAD, D), jnp.float32),
        pltpu.SemaphoreType.DMA,
    ]

    mesh = plsc.VectorSubcoreMesh(core_axis_name="c", subcore_axis_name="s")

    @functools.partial(
        pl.kernel, out_type=out_type, mesh=mesh, scratch_types=scratch,
    )
    def seg(x_hbm, src_hbm, dst_hbm, *refs):
        if with_cnt:
            agg_out, cnt_out, src_v, dst_v, rows_v, agg_sp, sem = refs
        else:
            agg_out, src_v, dst_v, rows_v, agg_sp, sem = refs
        cid = lax.axis_index("c")
        sid = lax.axis_index("s")
        wid = _wid(nc)
        row0 = sid * rows_per_tile
        n_sub = rows_per_tile // CHUNK
        base0 = wid * edges_per_tile

        def fill(val):
            def fill_body(k, carry):
                rows_v[k // 8, pl.ds((k % 8) * 16, 16)] = jnp.full(
                    (16,), val, jnp.float32)
                return carry

            lax.fori_loop(0, CHUNK * 8, fill_body, 0)

        def zero_stripe():
            for j in range(n_sub):
                pltpu.sync_copy(
                    rows_v, agg_sp.at[pl.ds(row0 + j * CHUNK, CHUNK)])

        def read_stripe(out):
            for j in range(n_sub):
                r = row0 + j * CHUNK
                pltpu.sync_copy(agg_sp.at[pl.ds(r, CHUNK)], rows_v)
                pltpu.sync_copy(rows_v, out.at[cid, pl.ds(r, CHUNK)])

        fill(0.0)
        zero_stripe()
        plsc.subcore_barrier()

        def body(k, carry):
            base = base0 + k * CHUNK
            pltpu.sync_copy(src_hbm.at[pl.ds(base, CHUNK)], src_v)
            pltpu.async_copy(x_hbm.at[src_v], rows_v, sem).wait()
            pltpu.sync_copy(dst_hbm.at[pl.ds(base, CHUNK)], dst_v)
            pltpu.sync_copy(rows_v, agg_sp.at[dst_v], add=True)
            return carry

        lax.fori_loop(0, n_chunks, body, 0)
        plsc.subcore_barrier()
        read_stripe(agg_out)

        if with_cnt:
            plsc.subcore_barrier()
            fill(0.0)
            zero_stripe()
            fill(1.0)
            plsc.subcore_barrier()

            def body2(k, carry):
                base = base0 + k * CHUNK
                pltpu.sync_copy(dst_hbm.at[pl.ds(base, CHUNK)], dst_v)
                pltpu.sync_copy(rows_v, agg_sp.at[dst_v], add=True)
                return carry

            lax.fori_loop(0, n_chunks, body2, 0)
            plsc.subcore_barrier()
            fill(0.0)
            read_stripe(cnt_out)

    return seg


_ROWS = 1024


def _tc_layer1(agg, cnt, x, wl, bl, wr):
    def body(agg_ref, cnt_ref, x_ref, wl_ref, bl_ref, wr_ref, out_ref):
        a = agg_ref[0] + agg_ref[1]
        c = cnt_ref[0, :, 0:1] + cnt_ref[1, :, 0:1]
        mean = a / jnp.maximum(c, 1.0)
        h = (
            jnp.dot(mean, wl_ref[...], preferred_element_type=jnp.float32)
            + bl_ref[...]
            + jnp.dot(x_ref[...], wr_ref[...], preferred_element_type=jnp.float32)
        )
        out_ref[...] = jnp.maximum(h, 0.0)

    grid = (N_PAD // _ROWS,)
    return pl.pallas_call(
        body,
        grid=grid,
        in_specs=[
            pl.BlockSpec((2, _ROWS, D), lambda i: (0, i, 0)),
            pl.BlockSpec((2, _ROWS, D), lambda i: (0, i, 0)),
            pl.BlockSpec((_ROWS, D), lambda i: (i, 0)),
            pl.BlockSpec((D, H), lambda i: (0, 0)),
            pl.BlockSpec((H,), lambda i: (0,)),
            pl.BlockSpec((D, H), lambda i: (0, 0)),
        ],
        out_specs=pl.BlockSpec((_ROWS, H), lambda i: (i, 0)),
        out_shape=jax.ShapeDtypeStruct((N_PAD, H), jnp.float32),
    )(agg, cnt, x, wl, bl, wr)


def _tc_layer2_decodeprep(agg, cnt, h1, wl, bl, wr, u, v, bp1):
    def body(agg_ref, cnt_ref, h1_ref, wl_ref, bl_ref, wr_ref, u_ref, v_ref,
             bp1_ref, a_out, b_out):
        a = agg_ref[0] + agg_ref[1]
        c = cnt_ref[0, :, 0:1] + cnt_ref[1, :, 0:1]
        mean = a / jnp.maximum(c, 1.0)
        h2 = (
            jnp.dot(mean, wl_ref[...], preferred_element_type=jnp.float32)
            + bl_ref[...]
            + jnp.dot(h1_ref[...], wr_ref[...], preferred_element_type=jnp.float32)
        )
        a_out[...] = (
            jnp.dot(h2, u_ref[...], preferred_element_type=jnp.float32)
            + bp1_ref[...]
        )
        b_out[...] = jnp.dot(h2, v_ref[...], preferred_element_type=jnp.float32)

    grid = (N_PAD // _ROWS,)
    return pl.pallas_call(
        body,
        grid=grid,
        in_specs=[
            pl.BlockSpec((2, _ROWS, H), lambda i: (0, i, 0)),
            pl.BlockSpec((2, _ROWS, D), lambda i: (0, i, 0)),
            pl.BlockSpec((_ROWS, H), lambda i: (i, 0)),
            pl.BlockSpec((H, H), lambda i: (0, 0)),
            pl.BlockSpec((H,), lambda i: (0,)),
            pl.BlockSpec((H, H), lambda i: (0, 0)),
            pl.BlockSpec((H, H), lambda i: (0, 0)),
            pl.BlockSpec((H, H), lambda i: (0, 0)),
            pl.BlockSpec((H,), lambda i: (0,)),
        ],
        out_specs=[
            pl.BlockSpec((_ROWS, H), lambda i: (i, 0)),
            pl.BlockSpec((_ROWS, H), lambda i: (i, 0)),
        ],
        out_shape=[
            jax.ShapeDtypeStruct((N_PAD, H), jnp.float32),
            jax.ShapeDtypeStruct((N_PAD, H), jnp.float32),
        ],
    )(agg, cnt, h1, wl, bl, wr, u, v, bp1)


def _make_decode_kernel(num_pairs, nc, ns):
    nw = nc * ns
    n_chunks = num_pairs // CHUNK
    rows_pc = CHUNK // 8
    max_chunks_per_w = -(-n_chunks // nw)

    mesh = plsc.VectorSubcoreMesh(core_axis_name="c", subcore_axis_name="s")

    @functools.partial(
        pl.kernel,
        out_type=jax.ShapeDtypeStruct((n_chunks, rows_pc, 128), jnp.float32),
        mesh=mesh,
        scratch_types=[
            pltpu.VMEM((CHUNK,), jnp.int32),
            pltpu.VMEM((CHUNK,), jnp.int32),
            pltpu.VMEM((CHUNK, H), jnp.float32),
            pltpu.VMEM((CHUNK, H), jnp.float32),
            pltpu.VMEM((H,), jnp.float32),
            pltpu.VMEM((rows_pc, 128), jnp.float32),
            pltpu.SemaphoreType.DMA,
        ],
    )
    def decode(a_hbm, b_hbm, ps_hbm, pd_hbm, wp2_hbm, out_hbm,
               ps_v, pd_v, arows_v, brows_v, wp2_v, out_v, sem):
        wid = _wid(nc)
        pltpu.sync_copy(wp2_hbm, wp2_v)

        def chunk_body(k, carry):
            ch = wid + k * nw

            @pl.when(ch < n_chunks)
            def _():
                base = ch * CHUNK
                pltpu.sync_copy(ps_hbm.at[pl.ds(base, CHUNK)], ps_v)
                pltpu.sync_copy(pd_hbm.at[pl.ds(base, CHUNK)], pd_v)
                pltpu.async_copy(a_hbm.at[ps_v], arows_v, sem).wait()
                pltpu.async_copy(b_hbm.at[pd_v], brows_v, sem).wait()

                def pair_body(p, carry2):
                    acc = jnp.zeros((16,), jnp.float32)
                    for j in range(H // 16):
                        va = (arows_v[p, pl.ds(j * 16, 16)]
                              + brows_v[p, pl.ds(j * 16, 16)])
                        va = jnp.maximum(va, 0.0)
                        acc = acc + va * wp2_v[pl.ds(j * 16, 16)]
                    out_v[p // 8, pl.ds((p % 8) * 16, 16)] = acc
                    return carry2

                lax.fori_loop(0, CHUNK, pair_body, 0)
                pltpu.sync_copy(out_v, out_hbm.at[ch])

            return carry

        lax.fori_loop(0, max_chunks_per_w, chunk_body, 0)

    return decode


def _tc_decode_fold(out2, bp2):
    num_rows = out2.shape[0]
    fold = jnp.repeat(jnp.eye(8, dtype=jnp.float32), 16, axis=0)

    def body(o2_ref, fold_ref, bp2_ref, out_ref):
        out_ref[...] = (
            jnp.dot(o2_ref[...], fold_ref[...],
                    preferred_element_type=jnp.float32)
            + bp2_ref[0, 0]
        )

    return pl.pallas_call(
        body,
        grid=(1,),
        in_specs=[
            pl.BlockSpec((num_rows, 128), lambda i: (0, 0)),
            pl.BlockSpec((128, 8), lambda i: (0, 0)),
            pl.BlockSpec((1, 1), lambda i: (0, 0), memory_space=pltpu.SMEM),
        ],
        out_specs=pl.BlockSpec((num_rows, 8), lambda i: (0, 0)),
        out_shape=jax.ShapeDtypeStruct((num_rows, 8), jnp.float32),
    )(out2, fold, bp2.reshape(1, 1))


def kernel(x, edge_index, edge_pairs, W1l, b1l, W1r, W2l, b2l, W2r, Wp1, bp1,
           Wp2, bp2):
    info = plsc.get_sparse_core_info()
    nc, ns = info.num_cores, info.num_subcores

    src = edge_index[0]
    dst = edge_index[1]
    ps = edge_pairs[0]
    pd = edge_pairs[1]
    num_edges = src.shape[0]
    num_pairs = ps.shape[0]

    x_pad = jnp.pad(x, ((0, N_PAD - N), (0, 0)))

    seg1 = _make_seg_kernel(num_edges, True, nc, ns)
    agg1, cnt = seg1(x_pad, src, dst)
    h1 = _tc_layer1(agg1, cnt, x_pad, W1l, b1l, W1r)

    seg2 = _make_seg_kernel(num_edges, False, nc, ns)
    (agg2,) = seg2(h1, src, dst)
    A, B = _tc_layer2_decodeprep(
        agg2, cnt, h1, W2l, b2l, W2r, Wp1[:H], Wp1[H:], bp1)

    decode = _make_decode_kernel(num_pairs, nc, ns)
    out2 = decode(A, B, ps, pd, Wp2[:, 0])
    folded = _tc_decode_fold(out2.reshape(num_pairs // 8, 128), bp2[0])
    return folded.reshape(num_pairs)


# speedup vs baseline: 5.9014x; 1.3838x over previous
import functools

import jax
import jax.numpy as jnp
from jax import lax
from jax.experimental import pallas as pl
from jax.experimental.pallas import tpu as pltpu
from jax.experimental.pallas import tpu_sc as plsc

N = 10000
N_PAD = 10240
D = 128
H = 128
CHUNK = 80


def _wid(nc):
    return lax.axis_index("s") * nc + lax.axis_index("c")


def _make_seg_kernel(num_edges, with_cnt, nc, ns):
    nw = nc * ns
    rows_per_tile = N_PAD // ns
    n_chunks = num_edges // (nw * CHUNK)
    edges_per_tile = num_edges // nw

    out_type = [jax.ShapeDtypeStruct((nc, N_PAD, D), jnp.float32)]
    if with_cnt:
        out_type.append(jax.ShapeDtypeStruct((nc, N_PAD, D), jnp.float32))
    scratch = [
        pltpu.VMEM((2, CHUNK), jnp.int32),
        pltpu.VMEM((2, CHUNK), jnp.int32),
        pltpu.VMEM((2, CHUNK, D), jnp.float32),
        pltpu.VMEM((CHUNK, D), jnp.float32),
        pltpu.VMEM_SHARED((N_PAD, D), jnp.float32),
        pltpu.SemaphoreType.DMA,
        pltpu.SemaphoreType.DMA,
    ]

    mesh = plsc.VectorSubcoreMesh(core_axis_name="c", subcore_axis_name="s")

    @functools.partial(
        pl.kernel, out_type=out_type, mesh=mesh, scratch_types=scratch,
    )
    def seg(x_hbm, src_hbm, dst_hbm, *refs):
        if with_cnt:
            (agg_out, cnt_out, src_v, dst_v, rows_v, stage_v, agg_sp,
             sem0, sem1) = refs
        else:
            agg_out, src_v, dst_v, rows_v, stage_v, agg_sp, sem0, sem1 = refs
        sems = (sem0, sem1)
        cid = lax.axis_index("c")
        sid = lax.axis_index("s")
        wid = _wid(nc)
        row0 = sid * rows_per_tile
        n_sub = rows_per_tile // CHUNK
        base0 = wid * edges_per_tile

        def fill(val):
            def fill_body(k, carry):
                stage_v[k // 8, pl.ds((k % 8) * 16, 16)] = jnp.full(
                    (16,), val, jnp.float32)
                return carry

            lax.fori_loop(0, CHUNK * 8, fill_body, 0)

        def zero_stripe():
            for j in range(n_sub):
                pltpu.sync_copy(
                    stage_v, agg_sp.at[pl.ds(row0 + j * CHUNK, CHUNK)])

        def read_stripe(out):
            for j in range(n_sub):
                r = row0 + j * CHUNK
                pltpu.sync_copy(agg_sp.at[pl.ds(r, CHUNK)], stage_v)
                pltpu.sync_copy(stage_v, out.at[cid, pl.ds(r, CHUNK)])

        def prefetch(k, b):
            base = base0 + k * CHUNK
            pltpu.sync_copy(src_hbm.at[pl.ds(base, CHUNK)], src_v.at[b])
            pltpu.sync_copy(dst_hbm.at[pl.ds(base, CHUNK)], dst_v.at[b])
            return pltpu.async_copy(
                x_hbm.at[src_v.at[b]], rows_v.at[b], sems[b])

        fill(0.0)
        zero_stripe()
        plsc.subcore_barrier()

        prefetch(0, 0)

        def body(k2, carry):
            for b in range(2):
                k = k2 * 2 + b

                @pl.when(k + 1 < n_chunks)
                def _():
                    prefetch(k + 1, 1 - b)

                @pl.when(k < n_chunks)
                def _():
                    pltpu.make_async_copy(
                        x_hbm.at[src_v.at[b]], rows_v.at[b], sems[b]).wait()
                    pltpu.sync_copy(
                        rows_v.at[b], agg_sp.at[dst_v.at[b]], add=True)
            return carry

        lax.fori_loop(0, (n_chunks + 2) // 2, body, 0)
        plsc.subcore_barrier()
        read_stripe(agg_out)

        if with_cnt:
            plsc.subcore_barrier()
            fill(0.0)
            zero_stripe()
            fill(1.0)
            plsc.subcore_barrier()

            def body2(k2, carry):
                for b in range(2):
                    k = k2 * 2 + b

                    @pl.when(k + 1 < n_chunks)
                    def _():
                        base = base0 + (k + 1) * CHUNK
                        pltpu.sync_copy(
                            dst_hbm.at[pl.ds(base, CHUNK)], dst_v.at[1 - b])

                    @pl.when(k < n_chunks)
                    def _():
                        pltpu.sync_copy(
                            stage_v, agg_sp.at[dst_v.at[b]], add=True)
                return carry

            base00 = base0
            pltpu.sync_copy(dst_hbm.at[pl.ds(base00, CHUNK)], dst_v.at[0])
            lax.fori_loop(0, (n_chunks + 2) // 2, body2, 0)
            plsc.subcore_barrier()
            fill(0.0)
            read_stripe(cnt_out)

    return seg


_ROWS = 1024


def _tc_layer1(agg, cnt, x, wl, bl, wr):
    def body(agg_ref, cnt_ref, x_ref, wl_ref, bl_ref, wr_ref, out_ref):
        a = agg_ref[0] + agg_ref[1]
        c = cnt_ref[0, :, 0:1] + cnt_ref[1, :, 0:1]
        mean = a / jnp.maximum(c, 1.0)
        h = (
            jnp.dot(mean, wl_ref[...], preferred_element_type=jnp.float32)
            + bl_ref[...]
            + jnp.dot(x_ref[...], wr_ref[...], preferred_element_type=jnp.float32)
        )
        out_ref[...] = jnp.maximum(h, 0.0)

    grid = (N_PAD // _ROWS,)
    return pl.pallas_call(
        body,
        grid=grid,
        in_specs=[
            pl.BlockSpec((2, _ROWS, D), lambda i: (0, i, 0)),
            pl.BlockSpec((2, _ROWS, D), lambda i: (0, i, 0)),
            pl.BlockSpec((_ROWS, D), lambda i: (i, 0)),
            pl.BlockSpec((D, H), lambda i: (0, 0)),
            pl.BlockSpec((H,), lambda i: (0,)),
            pl.BlockSpec((D, H), lambda i: (0, 0)),
        ],
        out_specs=pl.BlockSpec((_ROWS, H), lambda i: (i, 0)),
        out_shape=jax.ShapeDtypeStruct((N_PAD, H), jnp.float32),
    )(agg, cnt, x, wl, bl, wr)


def _tc_layer2_decodeprep(agg, cnt, h1, wl, bl, wr, u, v, bp1):
    def body(agg_ref, cnt_ref, h1_ref, wl_ref, bl_ref, wr_ref, u_ref, v_ref,
             bp1_ref, a_out, b_out):
        a = agg_ref[0] + agg_ref[1]
        c = cnt_ref[0, :, 0:1] + cnt_ref[1, :, 0:1]
        mean = a / jnp.maximum(c, 1.0)
        h2 = (
            jnp.dot(mean, wl_ref[...], preferred_element_type=jnp.float32)
            + bl_ref[...]
            + jnp.dot(h1_ref[...], wr_ref[...], preferred_element_type=jnp.float32)
        )
        a_out[...] = (
            jnp.dot(h2, u_ref[...], preferred_element_type=jnp.float32)
            + bp1_ref[...]
        )
        b_out[...] = jnp.dot(h2, v_ref[...], preferred_element_type=jnp.float32)

    grid = (N_PAD // _ROWS,)
    return pl.pallas_call(
        body,
        grid=grid,
        in_specs=[
            pl.BlockSpec((2, _ROWS, H), lambda i: (0, i, 0)),
            pl.BlockSpec((2, _ROWS, D), lambda i: (0, i, 0)),
            pl.BlockSpec((_ROWS, H), lambda i: (i, 0)),
            pl.BlockSpec((H, H), lambda i: (0, 0)),
            pl.BlockSpec((H,), lambda i: (0,)),
            pl.BlockSpec((H, H), lambda i: (0, 0)),
            pl.BlockSpec((H, H), lambda i: (0, 0)),
            pl.BlockSpec((H, H), lambda i: (0, 0)),
            pl.BlockSpec((H,), lambda i: (0,)),
        ],
        out_specs=[
            pl.BlockSpec((_ROWS, H), lambda i: (i, 0)),
            pl.BlockSpec((_ROWS, H), lambda i: (i, 0)),
        ],
        out_shape=[
            jax.ShapeDtypeStruct((N_PAD, H), jnp.float32),
            jax.ShapeDtypeStruct((N_PAD, H), jnp.float32),
        ],
    )(agg, cnt, h1, wl, bl, wr, u, v, bp1)


def _make_decode_kernel(num_pairs, nc, ns):
    nw = nc * ns
    n_chunks = num_pairs // CHUNK
    rows_pc = CHUNK // 8
    max_chunks_per_w = -(-n_chunks // nw)

    mesh = plsc.VectorSubcoreMesh(core_axis_name="c", subcore_axis_name="s")

    @functools.partial(
        pl.kernel,
        out_type=jax.ShapeDtypeStruct((n_chunks, rows_pc, 128), jnp.float32),
        mesh=mesh,
        scratch_types=[
            pltpu.VMEM((CHUNK,), jnp.int32),
            pltpu.VMEM((CHUNK,), jnp.int32),
            pltpu.VMEM((CHUNK, H), jnp.float32),
            pltpu.VMEM((CHUNK, H), jnp.float32),
            pltpu.VMEM((H,), jnp.float32),
            pltpu.VMEM((rows_pc, 128), jnp.float32),
            pltpu.SemaphoreType.DMA,
            pltpu.SemaphoreType.DMA,
        ],
    )
    def decode(a_hbm, b_hbm, ps_hbm, pd_hbm, wp2_hbm, out_hbm,
               ps_v, pd_v, arows_v, brows_v, wp2_v, out_v, sem, sem2):
        wid = _wid(nc)
        pltpu.sync_copy(wp2_hbm, wp2_v)

        def chunk_body(k, carry):
            ch = wid + k * nw

            @pl.when(ch < n_chunks)
            def _():
                base = ch * CHUNK
                pltpu.sync_copy(ps_hbm.at[pl.ds(base, CHUNK)], ps_v)
                pltpu.sync_copy(pd_hbm.at[pl.ds(base, CHUNK)], pd_v)
                ca = pltpu.async_copy(a_hbm.at[ps_v], arows_v, sem)
                cb = pltpu.async_copy(b_hbm.at[pd_v], brows_v, sem2)
                ca.wait()
                cb.wait()

                def pair_body(p, carry2):
                    acc = jnp.zeros((16,), jnp.float32)
                    for j in range(H // 16):
                        va = (arows_v[p, pl.ds(j * 16, 16)]
                              + brows_v[p, pl.ds(j * 16, 16)])
                        va = jnp.maximum(va, 0.0)
                        acc = acc + va * wp2_v[pl.ds(j * 16, 16)]
                    out_v[p // 8, pl.ds((p % 8) * 16, 16)] = acc
                    return carry2

                lax.fori_loop(0, CHUNK, pair_body, 0)
                pltpu.sync_copy(out_v, out_hbm.at[ch])

            return carry

        lax.fori_loop(0, max_chunks_per_w, chunk_body, 0)

    return decode


def _tc_decode_fold(out2, bp2):
    num_rows = out2.shape[0]
    fold = jnp.repeat(jnp.eye(8, dtype=jnp.float32), 16, axis=0)

    def body(o2_ref, fold_ref, bp2_ref, out_ref):
        out_ref[...] = (
            jnp.dot(o2_ref[...], fold_ref[...],
                    preferred_element_type=jnp.float32)
            + bp2_ref[0, 0]
        )

    return pl.pallas_call(
        body,
        grid=(1,),
        in_specs=[
            pl.BlockSpec((num_rows, 128), lambda i: (0, 0)),
            pl.BlockSpec((128, 8), lambda i: (0, 0)),
            pl.BlockSpec((1, 1), lambda i: (0, 0), memory_space=pltpu.SMEM),
        ],
        out_specs=pl.BlockSpec((num_rows, 8), lambda i: (0, 0)),
        out_shape=jax.ShapeDtypeStruct((num_rows, 8), jnp.float32),
    )(out2, fold, bp2.reshape(1, 1))


def kernel(x, edge_index, edge_pairs, W1l, b1l, W1r, W2l, b2l, W2r, Wp1, bp1,
           Wp2, bp2):
    info = plsc.get_sparse_core_info()
    nc, ns = info.num_cores, info.num_subcores

    src = edge_index[0]
    dst = edge_index[1]
    ps = edge_pairs[0]
    pd = edge_pairs[1]
    num_edges = src.shape[0]
    num_pairs = ps.shape[0]

    x_pad = jnp.pad(x, ((0, N_PAD - N), (0, 0)))

    seg1 = _make_seg_kernel(num_edges, True, nc, ns)
    agg1, cnt = seg1(x_pad, src, dst)
    h1 = _tc_layer1(agg1, cnt, x_pad, W1l, b1l, W1r)

    seg2 = _make_seg_kernel(num_edges, False, nc, ns)
    (agg2,) = seg2(h1, src, dst)
    A, B = _tc_layer2_decodeprep(
        agg2, cnt, h1, W2l, b2l, W2r, Wp1[:H], Wp1[H:], bp1)

    decode = _make_decode_kernel(num_pairs, nc, ns)
    out2 = decode(A, B, ps, pd, Wp2[:, 0])
    folded = _tc_decode_fold(out2.reshape(num_pairs // 8, 128), bp2[0])
    return folded.reshape(num_pairs)
